# Initial kernel scaffold; baseline (speedup 1.0000x reference)
import jax
import jax.numpy as jnp
from jax.experimental import pallas as pl


def kernel(actions, var_values, var_types, factor_ids, edge_attr_ids, senders, receivers, factor_emb, pred_emb, edge_emb, W_num, Wm_vf, Wu_f, Wm_fv, Wu_v, W_pol):
    raise NotImplementedError("write your pallas kernel here")


# speedup vs baseline: 4.1624x; 4.1624x over previous
import functools

import jax
import jax.numpy as jnp
from jax import lax
from jax.experimental import pallas as pl
from jax.experimental.pallas import tpu as pltpu
from jax.experimental.pallas import tpu_sc as plsc

N_VAR = 10000
N_FAC = 10000
E = 320000
D = 128
N_OBJ = 64
N_PRED = 128
ARITY = 8
N_ACT = 32
LAYERS = 3

NC = 2
NS = 16
NW = NC * NS
CH = 128
N_CHUNKS = -(-E // (CH * NW)) * NW
E_PAD = N_CHUNKS * CH
CHUNKS_PER_W = N_CHUNKS // NW
ACC_ROWS = 10112
ZROWS = ACC_ROWS // NS
DUMMY_ROW = N_FAC

R = 400
N_BLK = N_FAC // R


def _edge_pass(table, gath_pad, attr_pad, scat_pad, zeros):
    mesh = plsc.VectorSubcoreMesh(core_axis_name="c", subcore_axis_name="s",
                                  num_cores=NC)

    def body(table_r, gath_r, attr_r, scat_r, zeros_r, out_r,
             gath_v, attr_v, scat_v, key_v, rows_v, acc, sem):
        c = lax.axis_index("c")
        s = lax.axis_index("s")
        wid = c * NS + s

        pltpu.sync_copy(zeros_r.at[pl.ds(s * ZROWS, ZROWS)],
                        acc.at[pl.ds(s * ZROWS, ZROWS)])
        plsc.subcore_barrier()

        def chunk_body(k, carry):
            off = (k * NW + wid) * CH
            pltpu.sync_copy(gath_r.at[pl.ds(off, CH)], gath_v)
            pltpu.sync_copy(attr_r.at[pl.ds(off, CH)], attr_v)
            pltpu.sync_copy(scat_r.at[pl.ds(off, CH)], scat_v)
            for i in range(CH // 16):
                sl = pl.ds(i * 16, 16)
                key_v[sl] = gath_v[sl] * ARITY + attr_v[sl]
            pltpu.async_copy(table_r.at[key_v], rows_v, sem).wait()
            pltpu.sync_copy(rows_v, acc.at[scat_v], add=True)
            return carry

        lax.fori_loop(0, CHUNKS_PER_W, chunk_body, 0)
        plsc.subcore_barrier()

        pltpu.sync_copy(acc.at[pl.ds(s * ZROWS, ZROWS)],
                        out_r.at[c, pl.ds(s * ZROWS, ZROWS)])

    f = pl.kernel(
        body,
        out_type=jax.ShapeDtypeStruct((NC, ACC_ROWS, D), jnp.float32),
        mesh=mesh,
        scratch_types=[
            pltpu.VMEM((CH,), jnp.int32),
            pltpu.VMEM((CH,), jnp.int32),
            pltpu.VMEM((CH,), jnp.int32),
            pltpu.VMEM((CH,), jnp.int32),
            pltpu.VMEM((CH, D), jnp.float32),
            pltpu.VMEM_SHARED((ACC_ROWS, D), jnp.float32),
            pltpu.SemaphoreType.DMA,
        ],
    )
    return f(table, gath_pad, attr_pad, scat_pad, zeros)


def _relu(x):
    return jnp.maximum(x, 0.0)


def _build_table(x_blk, wm, eemb, t_ref):
    xw = jnp.dot(x_blk, wm, preferred_element_type=jnp.float32)
    ew = jnp.dot(eemb, wm, preferred_element_type=jnp.float32)
    for a in range(ARITY):
        t_ref[:, a, :] = _relu(xw + ew[a:a + 1, :])


def _init_kernel(vt_ref, vv_ref, fid_ref, pemb_ref, wn0_ref, wn1_ref,
                 femb_ref, wm0_ref, eemb_ref, var_ref, fac_ref, t_ref):
    vt = vt_ref[0, 0, :]
    vv = vv_ref[0, 0, :]
    fid = fid_ref[0, 0, :]
    oh_p = (vt[:, None] == lax.broadcasted_iota(jnp.int32, (R, N_PRED), 1)
            ).astype(jnp.float32)
    pm = jnp.dot(pemb_ref[...], wn0_ref[...], preferred_element_type=jnp.float32)
    var = _relu(jnp.dot(oh_p, pm, preferred_element_type=jnp.float32)
                + vv[:, None] * wn1_ref[0:1, :])
    oh_f = (fid[:, None] == lax.broadcasted_iota(jnp.int32, (R, N_OBJ), 1)
            ).astype(jnp.float32)
    fac = jnp.dot(oh_f, femb_ref[...], preferred_element_type=jnp.float32)
    var_ref[...] = var
    fac_ref[...] = fac
    _build_table(var, wm0_ref[...], eemb_ref[...], t_ref)


def _update_kernel(x_ref, a0_ref, a1_ref, wut_ref, wub_ref, wm_ref, eemb_ref,
                   x_new_ref, t_ref):
    x_new = _relu(jnp.dot(x_ref[...], wut_ref[...],
                          preferred_element_type=jnp.float32)
                  + jnp.dot(a0_ref[0] + a1_ref[0], wub_ref[...],
                            preferred_element_type=jnp.float32))
    x_new_ref[...] = x_new
    _build_table(x_new, wm_ref[...], eemb_ref[...], t_ref)


def _head_kernel(f_ref, a0_ref, a1_ref, wut_ref, wub_ref, wpol_ref, act_ref,
                 out_ref, m_ref, s_ref, v_ref):
    i = pl.program_id(0)

    @pl.when(i == 0)
    def _():
        m_ref[0] = -1e30
        s_ref[0] = 0.0
        v_ref[0] = 0.0

    f_new = _relu(jnp.dot(f_ref[...], wut_ref[...],
                          preferred_element_type=jnp.float32)
                  + jnp.dot(a0_ref[0] + a1_ref[0], wub_ref[...],
                            preferred_element_type=jnp.float32))
    logits = jnp.dot(f_new, wpol_ref[...], preferred_element_type=jnp.float32)

    m_old = m_ref[0]
    m_new = jnp.maximum(m_old, jnp.max(logits))
    s_ref[0] = s_ref[0] * jnp.exp(m_old - m_new) + jnp.sum(jnp.exp(logits - m_new))
    m_ref[0] = m_new

    a = act_ref[0]
    fac = a // N_ACT
    act = a % N_ACT
    rows = i * R + lax.broadcasted_iota(jnp.int32, (R, N_ACT), 0)
    cols = lax.broadcasted_iota(jnp.int32, (R, N_ACT), 1)
    sel = jnp.logical_and(rows == fac, cols == act)
    v_ref[0] += jnp.sum(jnp.where(sel, logits, 0.0))

    @pl.when(i == N_BLK - 1)
    def _():
        out_ref[0] = v_ref[0] - m_ref[0] - jnp.log(s_ref[0])


def _row_spec():
    return pl.BlockSpec((R, D), lambda i: (i, 0))


def _full(shape):
    return pl.BlockSpec(shape, lambda i: tuple(0 for _ in shape))


def _tc_init(var_types3, var_values3, factor_ids3, pred_emb, wn0, wn1,
             factor_emb, wm0, edge_emb):
    return pl.pallas_call(
        _init_kernel,
        grid=(N_BLK,),
        in_specs=[
            pl.BlockSpec((1, 1, R), lambda i: (i, 0, 0)),
            pl.BlockSpec((1, 1, R), lambda i: (i, 0, 0)),
            pl.BlockSpec((1, 1, R), lambda i: (i, 0, 0)),
            _full((N_PRED, D)),
            _full((D, D)),
            _full((1, D)),
            _full((N_OBJ, D)),
            _full((D, D)),
            _full((ARITY, D)),
        ],
        out_specs=[
            _row_spec(),
            _row_spec(),
            pl.BlockSpec((R, ARITY, D), lambda i: (i, 0, 0)),
        ],
        out_shape=[
            jax.ShapeDtypeStruct((N_VAR, D), jnp.float32),
            jax.ShapeDtypeStruct((N_FAC, D), jnp.float32),
            jax.ShapeDtypeStruct((N_VAR, ARITY, D), jnp.float32),
        ],
    )(var_types3, var_values3, factor_ids3, pred_emb, wn0, wn1,
      factor_emb, wm0, edge_emb)


def _tc_update(x, agg, wut, wub, wm, edge_emb, n_rows):
    return pl.pallas_call(
        _update_kernel,
        grid=(N_BLK,),
        in_specs=[
            _row_spec(),
            pl.BlockSpec((1, R, D), lambda i: (0, i, 0)),
            pl.BlockSpec((1, R, D), lambda i: (1, i, 0)),
            _full((D, D)),
            _full((D, D)),
            _full((D, D)),
            _full((ARITY, D)),
        ],
        out_specs=[
            _row_spec(),
            pl.BlockSpec((R, ARITY, D), lambda i: (i, 0, 0)),
        ],
        out_shape=[
            jax.ShapeDtypeStruct((n_rows, D), jnp.float32),
            jax.ShapeDtypeStruct((n_rows, ARITY, D), jnp.float32),
        ],
    )(x, agg, agg, wut, wub, wm, edge_emb)


def _tc_head(factors, agg, wut, wub, wpol, actions):
    return pl.pallas_call(
        _head_kernel,
        grid=(N_BLK,),
        in_specs=[
            _row_spec(),
            pl.BlockSpec((1, R, D), lambda i: (0, i, 0)),
            pl.BlockSpec((1, R, D), lambda i: (1, i, 0)),
            _full((D, D)),
            _full((D, D)),
            _full((D, N_ACT)),
            pl.BlockSpec(memory_space=pltpu.SMEM),
        ],
        out_specs=pl.BlockSpec(memory_space=pltpu.SMEM),
        out_shape=jax.ShapeDtypeStruct((1,), jnp.float32),
        scratch_shapes=[
            pltpu.SMEM((1,), jnp.float32),
            pltpu.SMEM((1,), jnp.float32),
            pltpu.SMEM((1,), jnp.float32),
        ],
    )(factors, agg, agg, wut, wub, wpol, actions)


def kernel(actions, var_values, var_types, factor_ids, edge_attr_ids, senders,
           receivers, factor_emb, pred_emb, edge_emb, W_num, Wm_vf, Wu_f,
           Wm_fv, Wu_v, W_pol):
    i32 = jnp.int32
    senders = senders.astype(i32)
    receivers = receivers.astype(i32)
    edge_attr_ids = edge_attr_ids.astype(i32)
    var_types = var_types.astype(i32)
    factor_ids = factor_ids.astype(i32)
    actions = actions.astype(i32)

    pad = E_PAD - E
    attr_p = jnp.pad(edge_attr_ids, (0, pad))
    gath_vf = jnp.pad(senders, (0, pad))
    scat_vf = jnp.pad(receivers, (0, pad), constant_values=DUMMY_ROW)
    gath_fv = jnp.pad(receivers, (0, pad))
    scat_fv = jnp.pad(senders, (0, pad), constant_values=DUMMY_ROW)
    zeros = jnp.zeros((ACC_ROWS, D), jnp.float32)

    var_types3 = var_types.reshape(N_BLK, 1, R)
    var_values3 = var_values.reshape(N_BLK, 1, R)
    factor_ids3 = factor_ids.reshape(N_BLK, 1, R)
    wn0 = W_num[:D]
    wn1 = W_num[D:]

    variables, factors, t_vf = _tc_init(
        var_types3, var_values3, factor_ids3, pred_emb, wn0, wn1,
        factor_emb, Wm_vf[0], edge_emb)

    for l in range(LAYERS):
        t_flat = t_vf.reshape(N_VAR * ARITY, D)
        agg_f = _edge_pass(t_flat, gath_vf, attr_p, scat_vf, zeros)
        if l < LAYERS - 1:
            factors, t_fv = _tc_update(
                factors, agg_f, Wu_f[l][:D], Wu_f[l][D:], Wm_fv[l],
                edge_emb, N_FAC)
            agg_v = _edge_pass(t_fv.reshape(N_FAC * ARITY, D),
                               gath_fv, attr_p, scat_fv, zeros)
            variables, t_vf = _tc_update(
                variables, agg_v, Wu_v[l][:D], Wu_v[l][D:], Wm_vf[l + 1],
                edge_emb, N_VAR)
        else:
            out = _tc_head(factors, agg_f, Wu_f[l][:D], Wu_f[l][D:],
                           W_pol, actions)
    return out


# speedup vs baseline: 4.3765x; 1.0514x over previous
import functools

import jax
import jax.numpy as jnp
from jax import lax
from jax.experimental import pallas as pl
from jax.experimental.pallas import tpu as pltpu
from jax.experimental.pallas import tpu_sc as plsc

N_VAR = 10000
N_FAC = 10000
E = 320000
D = 128
N_OBJ = 64
N_PRED = 128
ARITY = 8
N_ACT = 32
LAYERS = 3

NC = 2
NS = 16
NW = NC * NS
CH = 128
CPW = 80
HALF = CPW // 2
E_PAD = NW * CPW * CH
ACC_ROWS = 10112
ZROWS = ACC_ROWS // NS
DUMMY_ROW = N_FAC

R = 400
N_BLK = N_FAC // R


def _edge_pass(table, key3, scat3, zeros):
    mesh = plsc.VectorSubcoreMesh(core_axis_name="c", subcore_axis_name="s",
                                  num_cores=NC)

    def body(table_r, key_r, scat_r, zeros_r, out_r,
             key_v, scat_v, rows0, rows1, acc, sem0, sem1):
        c = lax.axis_index("c")
        s = lax.axis_index("s")
        wid = c * NS + s

        pltpu.sync_copy(zeros_r.at[pl.ds(s * ZROWS, ZROWS)],
                        acc.at[pl.ds(s * ZROWS, ZROWS)])
        plsc.subcore_barrier()

        rows = (rows0, rows1)
        sems = (sem0, sem1)

        for h in range(2):
            pltpu.sync_copy(key_r.at[wid, pl.ds(h * HALF, HALF)], key_v)
            pltpu.sync_copy(scat_r.at[wid, pl.ds(h * HALF, HALF)], scat_v)
            pltpu.async_copy(table_r.at[key_v.at[0]], rows0, sem0)

            def group(g, carry):
                for b in range(2):
                    k = g * 2 + b
                    nb = (b + 1) % 2

                    @pl.when(k + 1 < HALF)
                    def _():
                        pltpu.async_copy(table_r.at[key_v.at[k + 1]],
                                         rows[nb], sems[nb])

                    pltpu.make_async_copy(table_r.at[pl.ds(0, CH)],
                                          rows[b], sems[b]).wait()
                    pltpu.sync_copy(rows[b], acc.at[scat_v.at[k]], add=True)
                return carry

            lax.fori_loop(0, HALF // 2, group, 0)
        plsc.subcore_barrier()

        pltpu.sync_copy(acc.at[pl.ds(s * ZROWS, ZROWS)],
                        out_r.at[c, pl.ds(s * ZROWS, ZROWS)])

    f = pl.kernel(
        body,
        out_type=jax.ShapeDtypeStruct((NC, ACC_ROWS, D), jnp.float32),
        mesh=mesh,
        scratch_types=[
            pltpu.VMEM((HALF, CH), jnp.int32),
            pltpu.VMEM((HALF, CH), jnp.int32),
            pltpu.VMEM((CH, D), jnp.float32),
            pltpu.VMEM((CH, D), jnp.float32),
            pltpu.VMEM_SHARED((ACC_ROWS, D), jnp.float32),
            pltpu.SemaphoreType.DMA,
            pltpu.SemaphoreType.DMA,
        ],
    )
    return f(table, key3, scat3, zeros)


def _relu(x):
    return jnp.maximum(x, 0.0)


def _build_table(x_blk, wm, eemb, t_ref):
    xw = jnp.dot(x_blk, wm, preferred_element_type=jnp.float32)
    ew = jnp.dot(eemb, wm, preferred_element_type=jnp.float32)
    for a in range(ARITY):
        t_ref[:, a, :] = _relu(xw + ew[a:a + 1, :])


def _init_kernel(vt_ref, vv_ref, fid_ref, pemb_ref, wn0_ref, wn1_ref,
                 femb_ref, wm0_ref, eemb_ref, var_ref, fac_ref, t_ref):
    vt = vt_ref[0, 0, :]
    vv = vv_ref[0, 0, :]
    fid = fid_ref[0, 0, :]
    oh_p = (vt[:, None] == lax.broadcasted_iota(jnp.int32, (R, N_PRED), 1)
            ).astype(jnp.float32)
    pm = jnp.dot(pemb_ref[...], wn0_ref[...], preferred_element_type=jnp.float32)
    var = _relu(jnp.dot(oh_p, pm, preferred_element_type=jnp.float32)
                + vv[:, None] * wn1_ref[0:1, :])
    oh_f = (fid[:, None] == lax.broadcasted_iota(jnp.int32, (R, N_OBJ), 1)
            ).astype(jnp.float32)
    fac = jnp.dot(oh_f, femb_ref[...], preferred_element_type=jnp.float32)
    var_ref[...] = var
    fac_ref[...] = fac
    _build_table(var, wm0_ref[...], eemb_ref[...], t_ref)


def _update_kernel(x_ref, a0_ref, a1_ref, wut_ref, wub_ref, wm_ref, eemb_ref,
                   x_new_ref, t_ref):
    x_new = _relu(jnp.dot(x_ref[...], wut_ref[...],
                          preferred_element_type=jnp.float32)
                  + jnp.dot(a0_ref[0] + a1_ref[0], wub_ref[...],
                            preferred_element_type=jnp.float32))
    x_new_ref[...] = x_new
    _build_table(x_new, wm_ref[...], eemb_ref[...], t_ref)


def _head_kernel(f_ref, a0_ref, a1_ref, wut_ref, wub_ref, wpol_ref, act_ref,
                 out_ref, m_ref, s_ref, v_ref):
    i = pl.program_id(0)

    @pl.when(i == 0)
    def _():
        m_ref[0] = -1e30
        s_ref[0] = 0.0
        v_ref[0] = 0.0

    f_new = _relu(jnp.dot(f_ref[...], wut_ref[...],
                          preferred_element_type=jnp.float32)
                  + jnp.dot(a0_ref[0] + a1_ref[0], wub_ref[...],
                            preferred_element_type=jnp.float32))
    logits = jnp.dot(f_new, wpol_ref[...], preferred_element_type=jnp.float32)

    m_old = m_ref[0]
    m_new = jnp.maximum(m_old, jnp.max(logits))
    s_ref[0] = s_ref[0] * jnp.exp(m_old - m_new) + jnp.sum(jnp.exp(logits - m_new))
    m_ref[0] = m_new

    a = act_ref[0]
    fac = a // N_ACT
    act = a % N_ACT
    rows = i * R + lax.broadcasted_iota(jnp.int32, (R, N_ACT), 0)
    cols = lax.broadcasted_iota(jnp.int32, (R, N_ACT), 1)
    sel = jnp.logical_and(rows == fac, cols == act)
    v_ref[0] += jnp.sum(jnp.where(sel, logits, 0.0))

    @pl.when(i == N_BLK - 1)
    def _():
        out_ref[0] = v_ref[0] - m_ref[0] - jnp.log(s_ref[0])


def _row_spec():
    return pl.BlockSpec((R, D), lambda i: (i, 0))


def _full(shape):
    return pl.BlockSpec(shape, lambda i: tuple(0 for _ in shape))


def _tc_init(var_types3, var_values3, factor_ids3, pred_emb, wn0, wn1,
             factor_emb, wm0, edge_emb):
    return pl.pallas_call(
        _init_kernel,
        grid=(N_BLK,),
        in_specs=[
            pl.BlockSpec((1, 1, R), lambda i: (i, 0, 0)),
            pl.BlockSpec((1, 1, R), lambda i: (i, 0, 0)),
            pl.BlockSpec((1, 1, R), lambda i: (i, 0, 0)),
            _full((N_PRED, D)),
            _full((D, D)),
            _full((1, D)),
            _full((N_OBJ, D)),
            _full((D, D)),
            _full((ARITY, D)),
        ],
        out_specs=[
            _row_spec(),
            _row_spec(),
            pl.BlockSpec((R, ARITY, D), lambda i: (i, 0, 0)),
        ],
        out_shape=[
            jax.ShapeDtypeStruct((N_VAR, D), jnp.float32),
            jax.ShapeDtypeStruct((N_FAC, D), jnp.float32),
            jax.ShapeDtypeStruct((N_VAR, ARITY, D), jnp.float32),
        ],
    )(var_types3, var_values3, factor_ids3, pred_emb, wn0, wn1,
      factor_emb, wm0, edge_emb)


def _tc_update(x, agg, wut, wub, wm, edge_emb, n_rows):
    return pl.pallas_call(
        _update_kernel,
        grid=(N_BLK,),
        in_specs=[
            _row_spec(),
            pl.BlockSpec((1, R, D), lambda i: (0, i, 0)),
            pl.BlockSpec((1, R, D), lambda i: (1, i, 0)),
            _full((D, D)),
            _full((D, D)),
            _full((D, D)),
            _full((ARITY, D)),
        ],
        out_specs=[
            _row_spec(),
            pl.BlockSpec((R, ARITY, D), lambda i: (i, 0, 0)),
        ],
        out_shape=[
            jax.ShapeDtypeStruct((n_rows, D), jnp.float32),
            jax.ShapeDtypeStruct((n_rows, ARITY, D), jnp.float32),
        ],
    )(x, agg, agg, wut, wub, wm, edge_emb)


def _tc_head(factors, agg, wut, wub, wpol, actions):
    return pl.pallas_call(
        _head_kernel,
        grid=(N_BLK,),
        in_specs=[
            _row_spec(),
            pl.BlockSpec((1, R, D), lambda i: (0, i, 0)),
            pl.BlockSpec((1, R, D), lambda i: (1, i, 0)),
            _full((D, D)),
            _full((D, D)),
            _full((D, N_ACT)),
            pl.BlockSpec(memory_space=pltpu.SMEM),
        ],
        out_specs=pl.BlockSpec(memory_space=pltpu.SMEM),
        out_shape=jax.ShapeDtypeStruct((1,), jnp.float32),
        scratch_shapes=[
            pltpu.SMEM((1,), jnp.float32),
            pltpu.SMEM((1,), jnp.float32),
            pltpu.SMEM((1,), jnp.float32),
        ],
    )(factors, agg, agg, wut, wub, wpol, actions)


def kernel(actions, var_values, var_types, factor_ids, edge_attr_ids, senders,
           receivers, factor_emb, pred_emb, edge_emb, W_num, Wm_vf, Wu_f,
           Wm_fv, Wu_v, W_pol):
    i32 = jnp.int32
    senders = senders.astype(i32)
    receivers = receivers.astype(i32)
    edge_attr_ids = edge_attr_ids.astype(i32)
    var_types = var_types.astype(i32)
    factor_ids = factor_ids.astype(i32)
    actions = actions.astype(i32)

    pad = E_PAD - E

    def _lay(x, fill):
        return jnp.pad(x, (0, pad), constant_values=fill).reshape(NW, CPW, CH)

    key_vf = _lay(senders * ARITY + edge_attr_ids, 0)
    scat_vf = _lay(receivers, DUMMY_ROW)
    key_fv = _lay(receivers * ARITY + edge_attr_ids, 0)
    scat_fv = _lay(senders, DUMMY_ROW)
    zeros = jnp.zeros((ACC_ROWS, D), jnp.float32)

    var_types3 = var_types.reshape(N_BLK, 1, R)
    var_values3 = var_values.reshape(N_BLK, 1, R)
    factor_ids3 = factor_ids.reshape(N_BLK, 1, R)
    wn0 = W_num[:D]
    wn1 = W_num[D:]

    variables, factors, t_vf = _tc_init(
        var_types3, var_values3, factor_ids3, pred_emb, wn0, wn1,
        factor_emb, Wm_vf[0], edge_emb)

    for l in range(LAYERS):
        t_flat = t_vf.reshape(N_VAR * ARITY, D)
        agg_f = _edge_pass(t_flat, key_vf, scat_vf, zeros)
        if l < LAYERS - 1:
            factors, t_fv = _tc_update(
                factors, agg_f, Wu_f[l][:D], Wu_f[l][D:], Wm_fv[l],
                edge_emb, N_FAC)
            agg_v = _edge_pass(t_fv.reshape(N_FAC * ARITY, D),
                               key_fv, scat_fv, zeros)
            variables, t_vf = _tc_update(
                variables, agg_v, Wu_v[l][:D], Wu_v[l][D:], Wm_vf[l + 1],
                edge_emb, N_VAR)
        else:
            out = _tc_head(factors, agg_f, Wu_f[l][:D], Wu_f[l][D:],
                           W_pol, actions)
    return out
